# Initial kernel scaffold; baseline (speedup 1.0000x reference)
import jax
import jax.numpy as jnp
from jax.experimental import pallas as pl


def kernel(x, W1, b1, W2, b2):
    raise NotImplementedError("write your pallas kernel here")


# speedup vs baseline: 1.0408x; 1.0408x over previous
import functools

import jax
import jax.numpy as jnp
from jax.experimental import pallas as pl
from jax.experimental.pallas import tpu as pltpu

_R = 4096


def _gelu(z):
    c = 0.7978845608028654
    return 0.5 * z * (1.0 + jnp.tanh(c * (z + 0.044715 * z * z * z)))


def _body(x_ref, w1_ref, b1_ref, w2_ref, b2_ref, m10_ref,
          out_ref, mask_ref, pos_ref):
    xb = x_ref[...]

    nzf = (xb != 0.0).astype(jnp.float32) * m10_ref[...]
    cnt = jax.lax.dot_general(
        m10_ref[...], nzf, (((1,), (1,)), ((), ())),
        preferred_element_type=jnp.float32)
    maskv = (cnt == 0.0)
    mask_ref[...] = maskv.astype(jnp.uint8).reshape(1, 1, _R)
    cnt_row = jnp.sum(nzf, axis=1, keepdims=True)

    x7 = xb[:, :7]
    col7 = jax.lax.broadcasted_iota(jnp.int32, x7.shape, 1)
    pos = jnp.where(col7 < 4, x7, jnp.where(col7 == 5, 1.0, 0.0))
    pos_ref[...] = pos

    h = jnp.dot(xb, w1_ref[...], preferred_element_type=jnp.float32)
    h = _gelu(h + b1_ref[...])
    o = jnp.dot(h, w2_ref[...], preferred_element_type=jnp.float32)
    o = o + b2_ref[...]
    out_ref[...] = jnp.where(cnt_row != 0.0, o, 0.0)


@jax.jit
def _run(x2, W1, b1, W2, b2, m10):
    n = x2.shape[0]
    grid = (n // _R,)
    out, mask_u8, pos = pl.pallas_call(
        _body,
        grid=grid,
        in_specs=[
            pl.BlockSpec((_R, 32), lambda i: (i, 0)),
            pl.BlockSpec((32, 64), lambda i: (0, 0)),
            pl.BlockSpec((1, 64), lambda i: (0, 0)),
            pl.BlockSpec((64, 64), lambda i: (0, 0)),
            pl.BlockSpec((1, 64), lambda i: (0, 0)),
            pl.BlockSpec((1, 32), lambda i: (0, 0)),
        ],
        out_specs=[
            pl.BlockSpec((_R, 64), lambda i: (i, 0)),
            pl.BlockSpec((1, 1, _R), lambda i: (i, 0, 0)),
            pl.BlockSpec((_R, 7), lambda i: (i, 0)),
        ],
        out_shape=[
            jax.ShapeDtypeStruct((n, 64), jnp.float32),
            jax.ShapeDtypeStruct((n // _R, 1, _R), jnp.uint8),
            jax.ShapeDtypeStruct((n, 7), jnp.float32),
        ],
        compiler_params=pltpu.CompilerParams(
            dimension_semantics=("arbitrary",),
        ),
    )(x2, W1, b1, W2, b2, m10)
    return out, mask_u8, pos


def kernel(x, W1, b1, W2, b2):
    B, P, dim = x.shape
    hid = W2.shape[1]
    n = B * P
    x2 = x.reshape(n, dim)
    m10 = (jnp.arange(32, dtype=jnp.int32) < 10).astype(jnp.float32)[None, :]
    out, mask_u8, pos = _run(x2, W1, b1.reshape(1, hid), W2,
                             b2.reshape(1, hid), m10)
    return (out.reshape(B, P, hid),
            mask_u8.reshape(B, P).astype(jnp.bool_),
            pos.reshape(B, P, 7))
